# baseline (speedup 1.0000x reference)
import jax
import jax.numpy as jnp
from jax import lax
from jax.experimental import pallas as pl
from jax.experimental.pallas import tpu as pltpu
from jax.experimental.pallas import tpu_sc as plsc

VOCAB1 = 1000001
HIDDEN = 64
BATCH = 4096
SEQ = 200
NTOK = BATCH * SEQ
LANES = 128
NROWS = NTOK // LANES
NC, NS = 2, 16
NW = NC * NS
ROWS_PER_W = NROWS // NW
NB = 8
NCHUNK = ROWS_PER_W // NB

_sc_mesh = plsc.VectorSubcoreMesh(core_axis_name="c", subcore_axis_name="s")


def _sc_gather_body(tok_hbm, table_hbm, out_hbm, idx_v, rows_v, sem):
    wid = lax.axis_index("s") * NC + lax.axis_index("c")
    row0 = wid * ROWS_PER_W

    def chunk(c, carry):
        r = row0 + c * NB
        pltpu.sync_copy(tok_hbm.at[pl.ds(r, NB)], idx_v)
        cps = [
            pltpu.async_copy(table_hbm.at[idx_v.at[g]], rows_v.at[g], sem)
            for g in range(NB)
        ]
        for cp in cps:
            cp.wait()
        pltpu.sync_copy(rows_v, out_hbm.at[pl.ds(r, NB)])
        return carry

    lax.fori_loop(0, NCHUNK, chunk, 0)


_sc_gather = pl.kernel(
    _sc_gather_body,
    out_type=jax.ShapeDtypeStruct((NROWS, LANES, HIDDEN), jnp.float32),
    mesh=_sc_mesh,
    scratch_types=[
        pltpu.VMEM((NB, LANES), jnp.int32),
        pltpu.VMEM((NB, LANES, HIDDEN), jnp.float32),
        pltpu.SemaphoreType.DMA,
    ],
    compiler_params=pltpu.CompilerParams(use_tc_tiling_on_sc=False),
)

BB = 256
SCH = 8


def _tc_pool_body(emb_ref, w_ref, b_ref, out_ref):
    j = pl.program_id(1)
    x = emb_ref[...].reshape(BB * SCH, HIDDEN)
    y = lax.dot_general(
        x,
        w_ref[...],
        dimension_numbers=(((1,), (1,)), ((), ())),
        preferred_element_type=jnp.float32,
        precision=lax.Precision.HIGHEST,
    )
    m = jnp.max(y.reshape(BB, SCH, HIDDEN), axis=1) + b_ref[...]

    @pl.when(j == 0)
    def _():
        out_ref[...] = m

    @pl.when(j > 0)
    def _():
        out_ref[...] = jnp.maximum(out_ref[...], m)


def _tc_pool(emb, W, b2):
    return pl.pallas_call(
        _tc_pool_body,
        grid=(BATCH // BB, SEQ // SCH),
        in_specs=[
            pl.BlockSpec((BB, SCH, HIDDEN), lambda i, j: (i, j, 0)),
            pl.BlockSpec((HIDDEN, HIDDEN), lambda i, j: (0, 0)),
            pl.BlockSpec((1, HIDDEN), lambda i, j: (0, 0)),
        ],
        out_specs=pl.BlockSpec((BB, HIDDEN), lambda i, j: (i, 0)),
        out_shape=jax.ShapeDtypeStruct((BATCH, HIDDEN), jnp.float32),
    )(emb, W, b2)


def kernel(token_ids, table, W, b):
    tok = token_ids.astype(jnp.int32).reshape(NROWS, LANES)
    emb = _sc_gather(tok, table)
    emb = emb.reshape(BATCH, SEQ, HIDDEN)
    return _tc_pool(emb, W, b.reshape(1, HIDDEN))


# speedup vs baseline: 1.0649x; 1.0649x over previous
import jax
import jax.numpy as jnp
from jax import lax
from jax.experimental import pallas as pl
from jax.experimental.pallas import tpu as pltpu
from jax.experimental.pallas import tpu_sc as plsc

VOCAB1 = 1000001
HIDDEN = 64
BATCH = 4096
SEQ = 200
NTOK = BATCH * SEQ
LANES = 128
NROWS = NTOK // LANES
NC, NS = 2, 16
NW = NC * NS
ROWS_PER_W = NROWS // NW
NB = 8
NCHUNK = ROWS_PER_W // NB

_sc_mesh = plsc.VectorSubcoreMesh(core_axis_name="c", subcore_axis_name="s")


def _sc_gather_body(tok_hbm, table_hbm, out_hbm, idx_v, rows_v, sem):
    wid = lax.axis_index("s") * NC + lax.axis_index("c")
    row0 = wid * ROWS_PER_W

    def chunk(c, carry):
        r = row0 + c * NB
        pltpu.sync_copy(tok_hbm.at[pl.ds(r, NB)], idx_v)
        cps = [
            pltpu.async_copy(table_hbm.at[idx_v.at[g]], rows_v.at[g], sem)
            for g in range(NB)
        ]
        for cp in cps:
            cp.wait()
        pltpu.sync_copy(rows_v, out_hbm.at[pl.ds(r, NB)])
        return carry

    lax.fori_loop(0, NCHUNK, chunk, 0)


_sc_gather = pl.kernel(
    _sc_gather_body,
    out_type=jax.ShapeDtypeStruct((NROWS, LANES, HIDDEN), jnp.float32),
    mesh=_sc_mesh,
    scratch_types=[
        pltpu.VMEM((NB, LANES), jnp.int32),
        pltpu.VMEM((NB, LANES, HIDDEN), jnp.float32),
        pltpu.SemaphoreType.DMA,
    ],
    compiler_params=pltpu.CompilerParams(use_tc_tiling_on_sc=False),
)

BB = 16
RB = BB * SEQ // LANES


def _tc_pool_body(emb_ref, w_ref, b_ref, out_ref):
    x = emb_ref[...].reshape(BB * SEQ, HIDDEN)
    y = lax.dot_general(
        x,
        w_ref[...],
        dimension_numbers=(((1,), (1,)), ((), ())),
        preferred_element_type=jnp.float32,
        precision=lax.Precision.HIGHEST,
    )
    out_ref[...] = jnp.max(y.reshape(BB, SEQ, HIDDEN), axis=1) + b_ref[...]


def _tc_pool(emb, W, b2):
    return pl.pallas_call(
        _tc_pool_body,
        grid=(BATCH // BB,),
        in_specs=[
            pl.BlockSpec((RB, LANES, HIDDEN), lambda i: (i, 0, 0)),
            pl.BlockSpec((HIDDEN, HIDDEN), lambda i: (0, 0)),
            pl.BlockSpec((1, HIDDEN), lambda i: (0, 0)),
        ],
        out_specs=pl.BlockSpec((BB, HIDDEN), lambda i: (i, 0)),
        out_shape=jax.ShapeDtypeStruct((BATCH, HIDDEN), jnp.float32),
    )(emb, W, b2)


def kernel(token_ids, table, W, b):
    tok = token_ids.astype(jnp.int32).reshape(NROWS, LANES)
    emb = _sc_gather(tok, table)
    return _tc_pool(emb, W, b.reshape(1, HIDDEN))


# speedup vs baseline: 1.4912x; 1.4003x over previous
import jax
import jax.numpy as jnp
from jax import lax
from jax.experimental import pallas as pl
from jax.experimental.pallas import tpu as pltpu
from jax.experimental.pallas import tpu_sc as plsc

VOCAB1 = 1000001
HIDDEN = 64
BATCH = 4096
SEQ = 200
NTOK = BATCH * SEQ
LANES = 128
NROWS = NTOK // LANES
NC, NS = 2, 16
NW = NC * NS
ROWS_PER_W = NROWS // NW
NB = 8
NCHUNK = ROWS_PER_W // NB

_sc_mesh = plsc.VectorSubcoreMesh(core_axis_name="c", subcore_axis_name="s")


def _sc_gather_body(tok_hbm, table_hbm, out_hbm, idx_v, rows_v, sem):
    wid = lax.axis_index("s") * NC + lax.axis_index("c")
    row0 = wid * ROWS_PER_W

    def chunk(c, carry):
        r = row0 + c * NB
        pltpu.sync_copy(tok_hbm.at[pl.ds(r, NB)], idx_v)
        cps = [
            pltpu.async_copy(table_hbm.at[idx_v.at[g]], rows_v.at[g], sem)
            for g in range(NB)
        ]
        for cp in cps:
            cp.wait()
        pltpu.sync_copy(rows_v, out_hbm.at[pl.ds(r, NB)])
        return carry

    lax.fori_loop(0, NCHUNK, chunk, 0)


_sc_gather = pl.kernel(
    _sc_gather_body,
    out_type=jax.ShapeDtypeStruct((NROWS, LANES, HIDDEN), jnp.float32),
    mesh=_sc_mesh,
    scratch_types=[
        pltpu.VMEM((NB, LANES), jnp.int32),
        pltpu.VMEM((NB, LANES, HIDDEN), jnp.float32),
        pltpu.SemaphoreType.DMA,
    ],
    compiler_params=pltpu.CompilerParams(use_tc_tiling_on_sc=False),
)

BB = 16
PR = BB * SEQ // 2
H2 = 2 * HIDDEN


def _tc_pool_body(emb_ref, w_ref, b_ref, out_ref):
    y = lax.dot_general(
        emb_ref[...],
        w_ref[...],
        dimension_numbers=(((1,), (0,)), ((), ())),
        preferred_element_type=jnp.float32,
        precision=lax.Precision.HIGHEST,
    )
    m = jnp.max(y.reshape(BB, SEQ // 2, H2), axis=1)
    out_ref[...] = jnp.maximum(m[:, :HIDDEN], m[:, HIDDEN:]) + b_ref[...]


def _tc_pool(emb2, W2, b2):
    return pl.pallas_call(
        _tc_pool_body,
        grid=(BATCH // BB,),
        in_specs=[
            pl.BlockSpec((PR, H2), lambda i: (i, 0)),
            pl.BlockSpec((H2, H2), lambda i: (0, 0)),
            pl.BlockSpec((1, HIDDEN), lambda i: (0, 0)),
        ],
        out_specs=pl.BlockSpec((BB, HIDDEN), lambda i: (i, 0)),
        out_shape=jax.ShapeDtypeStruct((BATCH, HIDDEN), jnp.float32),
    )(emb2, W2, b2)


def kernel(token_ids, table, W, b):
    tok = token_ids.astype(jnp.int32).reshape(NROWS, LANES)
    emb = _sc_gather(tok, table)
    emb2 = emb.reshape(NTOK * HIDDEN // H2, H2)
    wt = W.T
    zero = jnp.zeros((HIDDEN, HIDDEN), dtype=jnp.float32)
    W2 = jnp.block([[wt, zero], [zero, wt]])
    return _tc_pool(emb2, W2, b.reshape(1, HIDDEN))
